# baseline (speedup 1.0000x reference)
import jax
import jax.numpy as jnp
from jax import lax
from jax.experimental import pallas as pl
from jax.experimental.pallas import tpu as pltpu
from jax.experimental.pallas import tpu_sc as plsc

R1 = 16
N = 100000
R2 = 16
B = 16384
RR = R1 * R2

NC = 2
NS = 16
NW = NC * NS
BPW = B // NW
CHUNK = 128
NCHUNK = BPW // CHUNK

NB = 8192
GRID1 = (N + NB - 1) // NB
BB = 2048
GRID3 = B // BB


def _k1_body(ct_ref, reg_ref, tab_ref):
    x = ct_ref[...]
    reg_ref[...] = x * x
    tab_ref[...] = x.T


@jax.jit
def _square_and_transpose(ct2):
    return pl.pallas_call(
        _k1_body,
        grid=(GRID1,),
        in_specs=[pl.BlockSpec((RR, NB), lambda i: (0, i))],
        out_specs=[
            pl.BlockSpec((RR, NB), lambda i: (0, i)),
            pl.BlockSpec((NB, RR), lambda i: (i, 0)),
        ],
        out_shape=[
            jax.ShapeDtypeStruct((RR, N), jnp.float32),
            jax.ShapeDtypeStruct((N, RR), jnp.float32),
        ],
    )(ct2)


def _gather_body(idx_hbm, tab_hbm, out_hbm, idx_v, buf0, buf1, sem0, sem1):
    wid = lax.axis_index("s") * NC + lax.axis_index("c")
    row0 = wid * NCHUNK
    pltpu.sync_copy(idx_hbm.at[pl.ds(row0, NCHUNK)], idx_v)

    bufs = (buf0, buf1)
    sems = (sem0, sem1)
    cps = [None, None]
    for j in range(NCHUNK):
        cps[j % 2] = pltpu.async_copy(
            tab_hbm.at[idx_v.at[j]], bufs[j % 2], sems[j % 2])
        if j > 0:
            cps[(j - 1) % 2].wait()
            base = wid * BPW + (j - 1) * CHUNK
            pltpu.sync_copy(bufs[(j - 1) % 2],
                            out_hbm.at[pl.ds(base, CHUNK)])
    cps[(NCHUNK - 1) % 2].wait()
    base = wid * BPW + (NCHUNK - 1) * CHUNK
    pltpu.sync_copy(bufs[(NCHUNK - 1) % 2], out_hbm.at[pl.ds(base, CHUNK)])


@jax.jit
def _gather(idx2d, tableT):
    mesh = plsc.VectorSubcoreMesh(
        core_axis_name="c", subcore_axis_name="s",
        num_cores=NC, num_subcores=NS)
    f = pl.kernel(
        _gather_body,
        out_type=jax.ShapeDtypeStruct((B, RR), jnp.float32),
        mesh=mesh,
        scratch_types=[
            pltpu.VMEM((NCHUNK, CHUNK), jnp.int32),
            pltpu.VMEM((CHUNK, RR), jnp.float32),
            pltpu.VMEM((CHUNK, RR), jnp.float32),
            pltpu.SemaphoreType.DMA,
            pltpu.SemaphoreType.DMA,
        ],
    )
    return f(idx2d, tableT)


def _k3_body(x_ref, o_ref):
    o_ref[...] = x_ref[...].T


@jax.jit
def _transpose_sel(sel_rm):
    return pl.pallas_call(
        _k3_body,
        grid=(GRID3,),
        in_specs=[pl.BlockSpec((BB, RR), lambda i: (i, 0))],
        out_specs=pl.BlockSpec((RR, BB), lambda i: (0, i)),
        out_shape=jax.ShapeDtypeStruct((RR, B), jnp.float32),
    )(sel_rm)


def kernel(indices, core_param):
    ct2 = jnp.transpose(core_param, (0, 2, 1)).reshape(RR, N)
    reg_t, tableT = _square_and_transpose(ct2)
    reg = jnp.transpose(reg_t.reshape(R1, R2, N), (0, 2, 1))

    idx2d = indices.reshape(B // CHUNK, CHUNK)
    sel_rm = _gather(idx2d, tableT)
    sel_t = _transpose_sel(sel_rm)
    sel = jnp.transpose(sel_t.reshape(R1, R2, B), (2, 0, 1))
    return (sel, reg)


# speedup vs baseline: 1.0070x; 1.0070x over previous
import jax
import jax.numpy as jnp
from jax import lax
from jax.experimental import pallas as pl
from jax.experimental.pallas import tpu as pltpu
from jax.experimental.pallas import tpu_sc as plsc

R1 = 16
N = 100000
R2 = 16
B = 16384
RR = R1 * R2

NC = 2
NS = 16
NW = NC * NS
BPW = B // NW
CHUNK = 128
NCHUNK = BPW // CHUNK

NB = 8192
GRID1 = (N + NB - 1) // NB
BB = 4096
GRID3 = B // BB


def _k1_body(ct_ref, reg_ref, tab_ref):
    x = ct_ref[...]
    reg_ref[...] = x * x
    tab_ref[...] = x.T


@jax.jit
def _square_and_transpose(ct2):
    return pl.pallas_call(
        _k1_body,
        grid=(GRID1,),
        in_specs=[pl.BlockSpec((RR, NB), lambda i: (0, i))],
        out_specs=[
            pl.BlockSpec((RR, NB), lambda i: (0, i)),
            pl.BlockSpec((NB, RR), lambda i: (i, 0)),
        ],
        out_shape=[
            jax.ShapeDtypeStruct((RR, N), jnp.float32),
            jax.ShapeDtypeStruct((N, RR), jnp.float32),
        ],
    )(ct2)


def _gather_body(idx_hbm, tab_hbm, out_hbm, idx_v, buf0, buf1, sem0, sem1):
    wid = lax.axis_index("s") * NC + lax.axis_index("c")
    row0 = wid * NCHUNK
    pltpu.sync_copy(idx_hbm.at[pl.ds(row0, NCHUNK)], idx_v)

    bufs = (buf0, buf1)
    sems = (sem0, sem1)
    cps = [None, None]
    for j in range(NCHUNK):
        cps[j % 2] = pltpu.async_copy(
            tab_hbm.at[idx_v.at[j]], bufs[j % 2], sems[j % 2])
        if j > 0:
            cps[(j - 1) % 2].wait()
            base = wid * BPW + (j - 1) * CHUNK
            pltpu.sync_copy(bufs[(j - 1) % 2],
                            out_hbm.at[pl.ds(base, CHUNK)])
    cps[(NCHUNK - 1) % 2].wait()
    base = wid * BPW + (NCHUNK - 1) * CHUNK
    pltpu.sync_copy(bufs[(NCHUNK - 1) % 2], out_hbm.at[pl.ds(base, CHUNK)])


@jax.jit
def _gather(idx2d, tableT):
    mesh = plsc.VectorSubcoreMesh(
        core_axis_name="c", subcore_axis_name="s",
        num_cores=NC, num_subcores=NS)
    f = pl.kernel(
        _gather_body,
        out_type=jax.ShapeDtypeStruct((B, RR), jnp.float32),
        mesh=mesh,
        scratch_types=[
            pltpu.VMEM((NCHUNK, CHUNK), jnp.int32),
            pltpu.VMEM((CHUNK, RR), jnp.float32),
            pltpu.VMEM((CHUNK, RR), jnp.float32),
            pltpu.SemaphoreType.DMA,
            pltpu.SemaphoreType.DMA,
        ],
    )
    return f(idx2d, tableT)


def _k3_body(x_ref, o_ref):
    o_ref[...] = x_ref[...].T


@jax.jit
def _transpose_sel(sel_rm):
    return pl.pallas_call(
        _k3_body,
        grid=(GRID3,),
        in_specs=[pl.BlockSpec((BB, RR), lambda i: (i, 0))],
        out_specs=pl.BlockSpec((RR, BB), lambda i: (0, i)),
        out_shape=jax.ShapeDtypeStruct((RR, B), jnp.float32),
    )(sel_rm)


def kernel(indices, core_param):
    ct2 = jnp.transpose(core_param, (0, 2, 1)).reshape(RR, N)
    reg_t, tableT = _square_and_transpose(ct2)
    reg = jnp.transpose(reg_t.reshape(R1, R2, N), (0, 2, 1))

    idx2d = indices.reshape(B // CHUNK, CHUNK)
    sel_rm = _gather(idx2d, tableT)
    sel_t = _transpose_sel(sel_rm)
    sel = jnp.transpose(sel_t.reshape(R1, R2, B), (2, 0, 1))
    return (sel, reg)


# speedup vs baseline: 1.0254x; 1.0183x over previous
import jax
import jax.numpy as jnp
from jax import lax
from jax.experimental import pallas as pl
from jax.experimental.pallas import tpu as pltpu
from jax.experimental.pallas import tpu_sc as plsc

R1 = 16
N = 100000
R2 = 16
B = 16384
RR = R1 * R2

NC = 2
NS = 16
NW = NC * NS
BPW = B // NW
CHUNK = 128
NCHUNK = BPW // CHUNK

NB = 8192
GRID1 = (N + NB - 1) // NB
BB = 8192
GRID3 = B // BB


def _k1_body(ct_ref, reg_ref, tab_ref):
    x = ct_ref[...]
    reg_ref[...] = x * x
    tab_ref[...] = x.T


@jax.jit
def _square_and_transpose(ct2):
    return pl.pallas_call(
        _k1_body,
        grid=(GRID1,),
        in_specs=[pl.BlockSpec((RR, NB), lambda i: (0, i))],
        out_specs=[
            pl.BlockSpec((RR, NB), lambda i: (0, i)),
            pl.BlockSpec((NB, RR), lambda i: (i, 0)),
        ],
        out_shape=[
            jax.ShapeDtypeStruct((RR, N), jnp.float32),
            jax.ShapeDtypeStruct((N, RR), jnp.float32),
        ],
    )(ct2)


NBUF = 3


def _gather_body(idx_hbm, tab_hbm, out_hbm, idx_v, buf0, buf1, buf2,
                 sem0, sem1, sem2):
    wid = lax.axis_index("s") * NC + lax.axis_index("c")
    row0 = wid * NCHUNK
    pltpu.sync_copy(idx_hbm.at[pl.ds(row0, NCHUNK)], idx_v)

    bufs = (buf0, buf1, buf2)
    sems = (sem0, sem1, sem2)
    cps = [None] * NBUF
    for j in range(NCHUNK):
        cps[j % NBUF] = pltpu.async_copy(
            tab_hbm.at[idx_v.at[j]], bufs[j % NBUF], sems[j % NBUF])
        if j >= NBUF - 1:
            k = j - (NBUF - 1)
            cps[k % NBUF].wait()
            pltpu.sync_copy(bufs[k % NBUF],
                            out_hbm.at[pl.ds(wid * BPW + k * CHUNK, CHUNK)])
    for k in range(NCHUNK - NBUF + 1, NCHUNK):
        cps[k % NBUF].wait()
        pltpu.sync_copy(bufs[k % NBUF],
                        out_hbm.at[pl.ds(wid * BPW + k * CHUNK, CHUNK)])


@jax.jit
def _gather(idx2d, tableT):
    mesh = plsc.VectorSubcoreMesh(
        core_axis_name="c", subcore_axis_name="s",
        num_cores=NC, num_subcores=NS)
    f = pl.kernel(
        _gather_body,
        out_type=jax.ShapeDtypeStruct((B, RR), jnp.float32),
        mesh=mesh,
        scratch_types=[
            pltpu.VMEM((NCHUNK, CHUNK), jnp.int32),
            pltpu.VMEM((CHUNK, RR), jnp.float32),
            pltpu.VMEM((CHUNK, RR), jnp.float32),
            pltpu.VMEM((CHUNK, RR), jnp.float32),
            pltpu.SemaphoreType.DMA,
            pltpu.SemaphoreType.DMA,
            pltpu.SemaphoreType.DMA,
        ],
    )
    return f(idx2d, tableT)


def _k3_body(x_ref, o_ref):
    o_ref[...] = x_ref[...].T


@jax.jit
def _transpose_sel(sel_rm):
    return pl.pallas_call(
        _k3_body,
        grid=(GRID3,),
        in_specs=[pl.BlockSpec((BB, RR), lambda i: (i, 0))],
        out_specs=pl.BlockSpec((RR, BB), lambda i: (0, i)),
        out_shape=jax.ShapeDtypeStruct((RR, B), jnp.float32),
    )(sel_rm)


def kernel(indices, core_param):
    ct2 = jnp.transpose(core_param, (0, 2, 1)).reshape(RR, N)
    reg_t, tableT = _square_and_transpose(ct2)
    reg = jnp.transpose(reg_t.reshape(R1, R2, N), (0, 2, 1))

    idx2d = indices.reshape(B // CHUNK, CHUNK)
    sel_rm = _gather(idx2d, tableT)
    sel_t = _transpose_sel(sel_rm)
    sel = jnp.transpose(sel_t.reshape(R1, R2, B), (2, 0, 1))
    return (sel, reg)


# speedup vs baseline: 1.0276x; 1.0021x over previous
import jax
import jax.numpy as jnp
from jax import lax
from jax.experimental import pallas as pl
from jax.experimental.pallas import tpu as pltpu
from jax.experimental.pallas import tpu_sc as plsc

R1 = 16
N = 100000
R2 = 16
B = 16384
RR = R1 * R2

NC = 2
NS = 16
NW = NC * NS
BPW = B // NW
CHUNK = 128
NCHUNK = BPW // CHUNK

NB = 10240
GRID1 = (N + NB - 1) // NB
BB = 8192
GRID3 = B // BB


def _k1_body(ct_ref, reg_ref, tab_ref):
    x = ct_ref[...]
    reg_ref[...] = x * x
    tab_ref[...] = x.T


@jax.jit
def _square_and_transpose(ct2):
    return pl.pallas_call(
        _k1_body,
        grid=(GRID1,),
        in_specs=[pl.BlockSpec((RR, NB), lambda i: (0, i))],
        out_specs=[
            pl.BlockSpec((RR, NB), lambda i: (0, i)),
            pl.BlockSpec((NB, RR), lambda i: (i, 0)),
        ],
        out_shape=[
            jax.ShapeDtypeStruct((RR, N), jnp.float32),
            jax.ShapeDtypeStruct((N, RR), jnp.float32),
        ],
        compiler_params=pltpu.CompilerParams(
            vmem_limit_bytes=62 * 1024 * 1024),
    )(ct2)


NBUF = 3


def _gather_body(idx_hbm, tab_hbm, out_hbm, idx_v, buf0, buf1, buf2,
                 sem0, sem1, sem2):
    wid = lax.axis_index("s") * NC + lax.axis_index("c")
    row0 = wid * NCHUNK
    pltpu.sync_copy(idx_hbm.at[pl.ds(row0, NCHUNK)], idx_v)

    bufs = (buf0, buf1, buf2)
    sems = (sem0, sem1, sem2)
    cps = [None] * NBUF
    for j in range(NCHUNK):
        cps[j % NBUF] = pltpu.async_copy(
            tab_hbm.at[idx_v.at[j]], bufs[j % NBUF], sems[j % NBUF])
        if j >= NBUF - 1:
            k = j - (NBUF - 1)
            cps[k % NBUF].wait()
            pltpu.sync_copy(bufs[k % NBUF],
                            out_hbm.at[pl.ds(wid * BPW + k * CHUNK, CHUNK)])
    for k in range(NCHUNK - NBUF + 1, NCHUNK):
        cps[k % NBUF].wait()
        pltpu.sync_copy(bufs[k % NBUF],
                        out_hbm.at[pl.ds(wid * BPW + k * CHUNK, CHUNK)])


@jax.jit
def _gather(idx2d, tableT):
    mesh = plsc.VectorSubcoreMesh(
        core_axis_name="c", subcore_axis_name="s",
        num_cores=NC, num_subcores=NS)
    f = pl.kernel(
        _gather_body,
        out_type=jax.ShapeDtypeStruct((B, RR), jnp.float32),
        mesh=mesh,
        scratch_types=[
            pltpu.VMEM((NCHUNK, CHUNK), jnp.int32),
            pltpu.VMEM((CHUNK, RR), jnp.float32),
            pltpu.VMEM((CHUNK, RR), jnp.float32),
            pltpu.VMEM((CHUNK, RR), jnp.float32),
            pltpu.SemaphoreType.DMA,
            pltpu.SemaphoreType.DMA,
            pltpu.SemaphoreType.DMA,
        ],
    )
    return f(idx2d, tableT)


def _k3_body(x_ref, o_ref):
    o_ref[...] = x_ref[...].T


@jax.jit
def _transpose_sel(sel_rm):
    return pl.pallas_call(
        _k3_body,
        grid=(GRID3,),
        in_specs=[pl.BlockSpec((BB, RR), lambda i: (i, 0))],
        out_specs=pl.BlockSpec((RR, BB), lambda i: (0, i)),
        out_shape=jax.ShapeDtypeStruct((RR, B), jnp.float32),
    )(sel_rm)


def kernel(indices, core_param):
    ct2 = jnp.transpose(core_param, (0, 2, 1)).reshape(RR, N)
    reg_t, tableT = _square_and_transpose(ct2)
    reg = jnp.transpose(reg_t.reshape(R1, R2, N), (0, 2, 1))

    idx2d = indices.reshape(B // CHUNK, CHUNK)
    sel_rm = _gather(idx2d, tableT)
    sel_t = _transpose_sel(sel_rm)
    sel = jnp.transpose(sel_t.reshape(R1, R2, B), (2, 0, 1))
    return (sel, reg)
